# baseline (speedup 1.0000x reference)
import functools

import jax
import jax.numpy as jnp
from jax import lax
from jax.experimental import pallas as pl
from jax.experimental.pallas import tpu as pltpu
from jax.experimental.pallas import tpu_sc as plsc

BATCH = 16384
EMB = 64
NTAB = 1000000
IDX_CHUNK = 128
WIN = 128
NWIN = (NTAB + WIN - 1) // WIN
WPT = (NWIN + 15) // 16
SENTINEL = 0x7FFFFFF
FLUSH = 16
CBUF = FLUSH + 16


def _phase1_sweep(ids_hbm, table, stage, ids_v, mid, mpos, wmid, wmpos,
                  win, colbuf, posb, scal, wsem, s0, s1, t):
    first_w = t * WPT
    end_w = jnp.minimum(first_w + WPT, NWIN)
    nwin = end_w - first_w
    iota16 = lax.iota(jnp.int32, 16)
    sems = [s0, s1]

    pltpu.sync_copy(ids_hbm, ids_v.at[pl.ds(0, BATCH)])

    def filt(k, off):
        v = ids_v[pl.ds(k * 16, 16)]
        wv = v >> 7
        msk = (wv >= first_w) & (wv < end_w)
        cnt = plsc.all_reduce_population_count(msk)[0]
        plsc.store_compressed(mid.at[pl.ds(off, 16)], v, mask=msk)
        plsc.store_compressed(mpos.at[pl.ds(off, 16)], k * 16 + iota16, mask=msk)
        return off + cnt

    moff = lax.fori_loop(0, BATCH // 16, filt, 0)
    sent = jnp.full((16,), SENTINEL, jnp.int32)
    mid[pl.ds(moff, 16)] = sent
    scal[0] = moff
    scal[1] = 0

    def fire(w_t, slot):
        @pl.when(w_t < nwin)
        def _():
            col0 = pl.multiple_of((first_w + w_t) * WIN, WIN)
            pltpu.async_copy(table.at[:, pl.ds(col0, WIN)], win.at[slot],
                             sems[slot])

    def flush_batch(n_cond):
        for q in range(CBUF):
            @pl.when(q < n_cond)
            def _():
                p = posb[q]
                pltpu.async_copy(colbuf.at[pl.ds(q * EMB, EMB)],
                                 stage.at[pl.ds(p * EMB, EMB)], wsem)
        for q in range(CBUF):
            @pl.when(q < n_cond)
            def _():
                pltpu.make_async_copy(colbuf.at[pl.ds(0, EMB)],
                                      stage.at[pl.ds(0, EMB)], wsem).wait()

    fire(0, 0)

    def outer(o, carry):
        for b in range(2):
            w_t = o * 2 + b
            fire(w_t + 1, (b + 1) % 2)

            @pl.when(w_t < nwin)
            def _process():
                gw = first_w + w_t
                pltpu.make_async_copy(table.at[:, pl.ds(0, WIN)],
                                      win.at[b], sems[b]).wait()

                @pl.when((w_t == 0) | ((gw & 15) == 0))
                def _rebuild():
                    gs = gw >> 4
                    mo = scal[0]

                    def rb(k, woff):
                        v = mid[pl.ds(k * 16, 16)]
                        pv = mpos[pl.ds(k * 16, 16)]
                        msk = (v >> 11) == gs
                        cnt = plsc.all_reduce_population_count(msk)[0]
                        plsc.store_compressed(wmid.at[pl.ds(woff, 16)], v,
                                              mask=msk)
                        plsc.store_compressed(wmpos.at[pl.ds(woff, 16)], pv,
                                              mask=msk)
                        return woff + cnt

                    woff = lax.fori_loop(0, (mo + 15) // 16, rb, 0)
                    wmid[pl.ds(woff, 16)] = sent
                    scal[2] = woff

                wo = scal[2]

                def scan(k, carry2):
                    v = wmid[pl.ds(k * 16, 16)]
                    pv = wmpos[pl.ds(k * 16, 16)]
                    msk = (v >> 7) == gw
                    m32 = jnp.where(msk, 1, 0)
                    nm = plsc.all_reduce_population_count(msk)[0]

                    @pl.when(nm > 0)
                    def _lanes():
                        for jj in range(16):
                            @pl.when(m32[jj] != 0)
                            def _one():
                                lane = v[jj] & (WIN - 1)
                                pos = pv[jj]
                                bc = scal[1]
                                lv = jnp.full((16,), 0, jnp.int32) + lane
                                for fb in range(EMB // 16):
                                    g16 = plsc.load_gather(
                                        win.at[b], [fb * 16 + iota16, lv])
                                    colbuf[pl.ds(bc * EMB + fb * 16, 16)] = g16
                                posb[bc] = pos
                                scal[1] = bc + 1

                        @pl.when(scal[1] >= FLUSH)
                        def _fl():
                            flush_batch(scal[1])
                            scal[1] = 0
                    return carry2

                lax.fori_loop(0, (wo + 15) // 16, scan, 0)
        return carry

    lax.fori_loop(0, (WPT + 1) // 2, outer, 0)
    flush_batch(scal[1])


def _phase1_body(u_id, i_id, uT, iT, u_stage, i_stage,
                 ids_v, mid, mpos, wmid, wmpos, win, colbuf,
                 posb, scal, wsem, s0, s1):
    c = lax.axis_index("c")
    s = lax.axis_index("s")

    @pl.when(c == 0)
    def _user():
        _phase1_sweep(u_id, uT, u_stage, ids_v, mid, mpos, wmid, wmpos,
                      win, colbuf, posb, scal, wsem, s0, s1, s)

    @pl.when(c == 1)
    def _item():
        _phase1_sweep(i_id, iT, i_stage, ids_v, mid, mpos, wmid, wmpos,
                      win, colbuf, posb, scal, wsem, s0, s1, s)


def _phase2_body(nw, bpw, u_id, i_id, u_stage, i_stage, ub, ib, mean, out,
                 uidx, iidx, u_v, i_v, bu, bi, mean_v, out_v, bsem):
    c = lax.axis_index("c")
    s = lax.axis_index("s")
    wid = s * 2 + c
    base = wid * bpw
    n_chunks = bpw // IDX_CHUNK

    pltpu.sync_copy(u_stage.at[pl.ds(base * EMB, bpw * EMB)], u_v)
    pltpu.sync_copy(i_stage.at[pl.ds(base * EMB, bpw * EMB)], i_v)
    for j in range(n_chunks):
        pltpu.sync_copy(u_id.at[pl.ds(base + j * IDX_CHUNK, IDX_CHUNK)], uidx.at[j])
        pltpu.sync_copy(i_id.at[pl.ds(base + j * IDX_CHUNK, IDX_CHUNK)], iidx.at[j])
    pltpu.sync_copy(mean, mean_v.at[pl.ds(0, 1)])

    bias_copies = []
    for j in range(n_chunks):
        sl = pl.ds(j * IDX_CHUNK, IDX_CHUNK)
        bias_copies.append(pltpu.async_copy(ub.at[uidx.at[j]], bu.at[sl], bsem))
        bias_copies.append(pltpu.async_copy(ib.at[iidx.at[j]], bi.at[sl], bsem))
    for cp in bias_copies:
        cp.wait()

    mean_s = mean_v[pl.ds(0, 16)][0]
    iota16 = lax.iota(jnp.int32, 16)

    def group(g, carry):
        o = g * 16
        fidx = (o + iota16) * EMB
        acc = bu[pl.ds(o, 16)] + bi[pl.ds(o, 16)] + mean_s
        for k in range(EMB):
            uu = plsc.load_gather(u_v, [fidx + k])
            ii = plsc.load_gather(i_v, [fidx + k])
            acc = acc + uu * ii
        out_v[pl.ds(o, 16)] = acc
        return carry

    lax.fori_loop(0, bpw // 16, group, 0)
    pltpu.sync_copy(out_v, out.at[pl.ds(base, bpw)])


def kernel(u_id, i_id, user_emb, user_bias, item_emb, item_bias, mean):
    info = plsc.get_sparse_core_info()
    nw = info.num_cores * info.num_subcores
    bpw = BATCH // nw
    mesh = plsc.VectorSubcoreMesh(core_axis_name="c", subcore_axis_name="s")
    params = pltpu.CompilerParams(needs_layout_passes=False,
                                  use_tc_tiling_on_sc=True)

    uid32 = u_id.astype(jnp.int32)
    iid32 = i_id.astype(jnp.int32)

    k1 = functools.partial(
        pl.kernel,
        mesh=mesh,
        out_type=(jax.ShapeDtypeStruct((BATCH * EMB,), jnp.float32),
                  jax.ShapeDtypeStruct((BATCH * EMB,), jnp.float32)),
        compiler_params=params,
        scratch_types=[
            pltpu.VMEM((BATCH,), jnp.int32),
            pltpu.VMEM((BATCH + 16,), jnp.int32),
            pltpu.VMEM((BATCH + 16,), jnp.int32),
            pltpu.VMEM((BATCH + 16,), jnp.int32),
            pltpu.VMEM((BATCH + 16,), jnp.int32),
            pltpu.VMEM((2, EMB, WIN), jnp.float32),
            pltpu.VMEM((CBUF * EMB,), jnp.float32),
            pltpu.SMEM((CBUF,), jnp.int32),
            pltpu.SMEM((8,), jnp.int32),
            pltpu.SemaphoreType.DMA,
            pltpu.SemaphoreType.DMA,
            pltpu.SemaphoreType.DMA,
        ],
    )(_phase1_body)

    u_stage, i_stage = k1(uid32, iid32, user_emb.T, item_emb.T)

    k2 = functools.partial(
        pl.kernel,
        mesh=mesh,
        out_type=jax.ShapeDtypeStruct((BATCH,), jnp.float32),
        compiler_params=params,
        scratch_types=[
            pltpu.VMEM((bpw // IDX_CHUNK, IDX_CHUNK), jnp.int32),
            pltpu.VMEM((bpw // IDX_CHUNK, IDX_CHUNK), jnp.int32),
            pltpu.VMEM((bpw * EMB,), jnp.float32),
            pltpu.VMEM((bpw * EMB,), jnp.float32),
            pltpu.VMEM((bpw,), jnp.float32),
            pltpu.VMEM((bpw,), jnp.float32),
            pltpu.VMEM((16,), jnp.float32),
            pltpu.VMEM((bpw,), jnp.float32),
            pltpu.SemaphoreType.DMA,
        ],
    )(functools.partial(_phase2_body, nw, bpw))

    return k2(uid32, iid32, u_stage, i_stage,
              user_bias.reshape(-1), item_bias.reshape(-1), mean)


# speedup vs baseline: 1.3390x; 1.3390x over previous
import functools

import jax
import jax.numpy as jnp
from jax import lax
from jax.experimental import pallas as pl
from jax.experimental.pallas import tpu as pltpu
from jax.experimental.pallas import tpu_sc as plsc

BATCH = 16384
EMB = 64
NTAB = 1000000
IDX_CHUNK = 128
WIN = 128
NWIN = (NTAB + WIN - 1) // WIN
WPT = (NWIN + 15) // 16
FLUSH = 16
CBUF = FLUSH + 16


def _phase1_sweep(ids_hbm, table, stage, ids_v, mid, mpos, sid, spos,
                  win, colbuf, posb, scal, cnt_s, wsem, s0, s1, t):
    first_w = t * WPT
    end_w = jnp.minimum(first_w + WPT, NWIN)
    nwin = end_w - first_w
    iota16 = lax.iota(jnp.int32, 16)
    lane0 = iota16 == 0
    sems = [s0, s1]

    pltpu.sync_copy(ids_hbm, ids_v.at[pl.ds(0, BATCH)])

    def filt(k, off):
        v = ids_v[pl.ds(k * 16, 16)]
        wv = v >> 7
        msk = (wv >= first_w) & (wv < end_w)
        cnt = plsc.all_reduce_population_count(msk)[0]
        plsc.store_compressed(mid.at[pl.ds(off, 16)], v, mask=msk)
        plsc.store_compressed(mpos.at[pl.ds(off, 16)], k * 16 + iota16, mask=msk)
        return off + cnt

    moff = lax.fori_loop(0, BATCH // 16, filt, 0)
    scal[0] = moff
    scal[1] = 0

    def czero(w, carry):
        cnt_s[w] = 0
        return carry

    lax.fori_loop(0, WPT, czero, 0)

    def count(k, carry):
        v = mid[pl.ds(k * 16, 16)]
        for jj in range(16):
            @pl.when(k * 16 + jj < moff)
            def _c():
                w = (v[jj] >> 7) - first_w
                cnt_s[w] = cnt_s[w] + 1
        return carry

    lax.fori_loop(0, (moff + 15) // 16, count, 0)

    def prefix(w, acc):
        c = cnt_s[w]
        cnt_s[w] = acc
        return acc + c

    lax.fori_loop(0, WPT, prefix, 0)

    def place(k, carry):
        v = mid[pl.ds(k * 16, 16)]
        pv = mpos[pl.ds(k * 16, 16)]
        for jj in range(16):
            @pl.when(k * 16 + jj < moff)
            def _p():
                w = (v[jj] >> 7) - first_w
                slot = cnt_s[w]
                sv = jnp.full((16,), 0, jnp.int32) + slot
                plsc.store_scatter(sid, [sv], jnp.full((16,), 0, jnp.int32) + v[jj],
                                   mask=lane0)
                plsc.store_scatter(spos, [sv], jnp.full((16,), 0, jnp.int32) + pv[jj],
                                   mask=lane0)
                cnt_s[w] = slot + 1
        return carry

    lax.fori_loop(0, (moff + 15) // 16, place, 0)

    def flush_batch(n_cond):
        for q in range(CBUF):
            @pl.when(q < n_cond)
            def _():
                p = posb[q]
                pltpu.async_copy(colbuf.at[pl.ds(q * EMB, EMB)],
                                 stage.at[pl.ds(p * EMB, EMB)], wsem)
        for q in range(CBUF):
            @pl.when(q < n_cond)
            def _():
                pltpu.make_async_copy(colbuf.at[pl.ds(0, EMB)],
                                      stage.at[pl.ds(0, EMB)], wsem).wait()

    def fire(w_t, slot):
        @pl.when(w_t < nwin)
        def _():
            col0 = pl.multiple_of((first_w + w_t) * WIN, WIN)
            pltpu.async_copy(table.at[:, pl.ds(col0, WIN)], win.at[slot],
                             sems[slot])

    fire(0, 0)

    def outer(o, carry):
        for b in range(2):
            w_t = o * 2 + b
            fire(w_t + 1, (b + 1) % 2)

            @pl.when(w_t < nwin)
            def _process():
                pltpu.make_async_copy(table.at[:, pl.ds(0, WIN)],
                                      win.at[b], sems[b]).wait()
                start = jnp.where(w_t > 0, cnt_s[jnp.maximum(w_t - 1, 0)], 0)
                end = cnt_s[w_t]

                def scan(k, carry2):
                    v = sid[pl.ds(start + k * 16, 16)]
                    pv = spos[pl.ds(start + k * 16, 16)]
                    for jj in range(16):
                        @pl.when(start + k * 16 + jj < end)
                        def _one():
                            lane = v[jj] & (WIN - 1)
                            pos = pv[jj]
                            bc = scal[1]
                            lv = jnp.full((16,), 0, jnp.int32) + lane
                            for fb in range(EMB // 16):
                                g16 = plsc.load_gather(
                                    win.at[b], [fb * 16 + iota16, lv])
                                colbuf[pl.ds(bc * EMB + fb * 16, 16)] = g16
                            posb[bc] = pos
                            scal[1] = bc + 1

                    @pl.when(scal[1] >= FLUSH)
                    def _fl():
                        flush_batch(scal[1])
                        scal[1] = 0
                    return carry2

                lax.fori_loop(0, (end - start + 15) // 16, scan, 0)
        return carry

    lax.fori_loop(0, (WPT + 1) // 2, outer, 0)
    flush_batch(scal[1])


def _phase1_body(u_id, i_id, uT, iT, u_stage, i_stage,
                 ids_v, mid, mpos, sid, spos, win, colbuf,
                 posb, scal, cnt_s, wsem, s0, s1):
    c = lax.axis_index("c")
    s = lax.axis_index("s")

    @pl.when(c == 0)
    def _user():
        _phase1_sweep(u_id, uT, u_stage, ids_v, mid, mpos, sid, spos,
                      win, colbuf, posb, scal, cnt_s, wsem, s0, s1, s)

    @pl.when(c == 1)
    def _item():
        _phase1_sweep(i_id, iT, i_stage, ids_v, mid, mpos, sid, spos,
                      win, colbuf, posb, scal, cnt_s, wsem, s0, s1, s)


def _phase2_body(nw, bpw, u_id, i_id, u_stage, i_stage, ub, ib, mean, out,
                 uidx, iidx, u_v, i_v, bu, bi, mean_v, out_v, bsem):
    c = lax.axis_index("c")
    s = lax.axis_index("s")
    wid = s * 2 + c
    base = wid * bpw
    n_chunks = bpw // IDX_CHUNK

    pltpu.sync_copy(u_stage.at[pl.ds(base * EMB, bpw * EMB)], u_v)
    pltpu.sync_copy(i_stage.at[pl.ds(base * EMB, bpw * EMB)], i_v)
    for j in range(n_chunks):
        pltpu.sync_copy(u_id.at[pl.ds(base + j * IDX_CHUNK, IDX_CHUNK)], uidx.at[j])
        pltpu.sync_copy(i_id.at[pl.ds(base + j * IDX_CHUNK, IDX_CHUNK)], iidx.at[j])
    pltpu.sync_copy(mean, mean_v.at[pl.ds(0, 1)])

    bias_copies = []
    for j in range(n_chunks):
        sl = pl.ds(j * IDX_CHUNK, IDX_CHUNK)
        bias_copies.append(pltpu.async_copy(ub.at[uidx.at[j]], bu.at[sl], bsem))
        bias_copies.append(pltpu.async_copy(ib.at[iidx.at[j]], bi.at[sl], bsem))
    for cp in bias_copies:
        cp.wait()

    mean_s = mean_v[pl.ds(0, 16)][0]
    iota16 = lax.iota(jnp.int32, 16)

    def group(g, carry):
        o = g * 16
        fidx = (o + iota16) * EMB
        acc = bu[pl.ds(o, 16)] + bi[pl.ds(o, 16)] + mean_s
        for k in range(EMB):
            uu = plsc.load_gather(u_v, [fidx + k])
            ii = plsc.load_gather(i_v, [fidx + k])
            acc = acc + uu * ii
        out_v[pl.ds(o, 16)] = acc
        return carry

    lax.fori_loop(0, bpw // 16, group, 0)
    pltpu.sync_copy(out_v, out.at[pl.ds(base, bpw)])


def kernel(u_id, i_id, user_emb, user_bias, item_emb, item_bias, mean):
    info = plsc.get_sparse_core_info()
    nw = info.num_cores * info.num_subcores
    bpw = BATCH // nw
    mesh = plsc.VectorSubcoreMesh(core_axis_name="c", subcore_axis_name="s")
    params = pltpu.CompilerParams(needs_layout_passes=False,
                                  use_tc_tiling_on_sc=True)

    uid32 = u_id.astype(jnp.int32)
    iid32 = i_id.astype(jnp.int32)

    k1 = functools.partial(
        pl.kernel,
        mesh=mesh,
        out_type=(jax.ShapeDtypeStruct((BATCH * EMB,), jnp.float32),
                  jax.ShapeDtypeStruct((BATCH * EMB,), jnp.float32)),
        compiler_params=params,
        scratch_types=[
            pltpu.VMEM((BATCH,), jnp.int32),
            pltpu.VMEM((BATCH + 16,), jnp.int32),
            pltpu.VMEM((BATCH + 16,), jnp.int32),
            pltpu.VMEM((BATCH + 16,), jnp.int32),
            pltpu.VMEM((BATCH + 16,), jnp.int32),
            pltpu.VMEM((2, EMB, WIN), jnp.float32),
            pltpu.VMEM((CBUF * EMB,), jnp.float32),
            pltpu.SMEM((CBUF,), jnp.int32),
            pltpu.SMEM((8,), jnp.int32),
            pltpu.SMEM((WPT,), jnp.int32),
            pltpu.SemaphoreType.DMA,
            pltpu.SemaphoreType.DMA,
            pltpu.SemaphoreType.DMA,
        ],
    )(_phase1_body)

    u_stage, i_stage = k1(uid32, iid32, user_emb.T, item_emb.T)

    k2 = functools.partial(
        pl.kernel,
        mesh=mesh,
        out_type=jax.ShapeDtypeStruct((BATCH,), jnp.float32),
        compiler_params=params,
        scratch_types=[
            pltpu.VMEM((bpw // IDX_CHUNK, IDX_CHUNK), jnp.int32),
            pltpu.VMEM((bpw // IDX_CHUNK, IDX_CHUNK), jnp.int32),
            pltpu.VMEM((bpw * EMB,), jnp.float32),
            pltpu.VMEM((bpw * EMB,), jnp.float32),
            pltpu.VMEM((bpw,), jnp.float32),
            pltpu.VMEM((bpw,), jnp.float32),
            pltpu.VMEM((16,), jnp.float32),
            pltpu.VMEM((bpw,), jnp.float32),
            pltpu.SemaphoreType.DMA,
        ],
    )(functools.partial(_phase2_body, nw, bpw))

    return k2(uid32, iid32, u_stage, i_stage,
              user_bias.reshape(-1), item_bias.reshape(-1), mean)


# speedup vs baseline: 1.9236x; 1.4366x over previous
import functools

import jax
import jax.numpy as jnp
from jax import lax
from jax.experimental import pallas as pl
from jax.experimental.pallas import tpu as pltpu
from jax.experimental.pallas import tpu_sc as plsc

BATCH = 16384
EMB = 64
NTAB = 1000000
IDX_CHUNK = 128
WIN = 128
NWIN = (NTAB + WIN - 1) // WIN
WPT = (NWIN + 15) // 16
FLUSH = 16
CBUF = FLUSH + 16
NRING = 4


def _phase1_sweep(ids_hbm, table, stage, ids_v, mid, mpos, sid, spos,
                  win, colbuf, posb, scal, cnt_s, wsem, wsems, t):
    first_w = t * WPT
    end_w = jnp.minimum(first_w + WPT, NWIN)
    nwin = end_w - first_w
    iota16 = lax.iota(jnp.int32, 16)
    lane0 = iota16 == 0

    pltpu.sync_copy(ids_hbm, ids_v.at[pl.ds(0, BATCH)])

    def filt(k, off):
        v = ids_v[pl.ds(k * 16, 16)]
        wv = v >> 7
        msk = (wv >= first_w) & (wv < end_w)
        cnt = plsc.all_reduce_population_count(msk)[0]
        plsc.store_compressed(mid.at[pl.ds(off, 16)], v, mask=msk)
        plsc.store_compressed(mpos.at[pl.ds(off, 16)], k * 16 + iota16, mask=msk)
        return off + cnt

    moff = lax.fori_loop(0, BATCH // 16, filt, 0)
    scal[0] = moff
    scal[1] = 0

    def czero(w, carry):
        cnt_s[w] = 0
        return carry

    lax.fori_loop(0, WPT, czero, 0)

    def count(k, carry):
        v = mid[pl.ds(k * 16, 16)]
        for jj in range(16):
            @pl.when(k * 16 + jj < moff)
            def _c():
                w = (v[jj] >> 7) - first_w
                cnt_s[w] = cnt_s[w] + 1
        return carry

    lax.fori_loop(0, (moff + 15) // 16, count, 0)

    def prefix(w, acc):
        c = cnt_s[w]
        cnt_s[w] = acc
        return acc + c

    lax.fori_loop(0, WPT, prefix, 0)

    def place(k, carry):
        v = mid[pl.ds(k * 16, 16)]
        pv = mpos[pl.ds(k * 16, 16)]
        for jj in range(16):
            @pl.when(k * 16 + jj < moff)
            def _p():
                w = (v[jj] >> 7) - first_w
                slot = cnt_s[w]
                sv = jnp.full((16,), 0, jnp.int32) + slot
                plsc.store_scatter(sid, [sv], jnp.full((16,), 0, jnp.int32) + v[jj],
                                   mask=lane0)
                plsc.store_scatter(spos, [sv], jnp.full((16,), 0, jnp.int32) + pv[jj],
                                   mask=lane0)
                cnt_s[w] = slot + 1
        return carry

    lax.fori_loop(0, (moff + 15) // 16, place, 0)

    def flush_batch(n_cond):
        for q in range(CBUF):
            @pl.when(q < n_cond)
            def _():
                p = posb[q]
                pltpu.async_copy(colbuf.at[pl.ds(q * EMB, EMB)],
                                 stage.at[pl.ds(p * EMB, EMB)], wsem)
        for q in range(CBUF):
            @pl.when(q < n_cond)
            def _():
                pltpu.make_async_copy(colbuf.at[pl.ds(0, EMB)],
                                      stage.at[pl.ds(0, EMB)], wsem).wait()

    def fire(w_t):
        @pl.when(w_t < nwin)
        def _():
            slot = w_t % NRING
            col0 = pl.multiple_of((first_w + w_t) * WIN, WIN)
            pltpu.async_copy(table.at[:, pl.ds(col0, WIN)], win.at[slot],
                             wsems.at[slot])

    for p in range(NRING - 1):
        fire(p)

    def outer(w, carry):
        fire(w + NRING - 1)
        w_t = w
        b = w_t % NRING

        @pl.when(w_t < nwin)
        def _process():
            pltpu.make_async_copy(table.at[:, pl.ds(0, WIN)],
                                  win.at[b], wsems.at[b]).wait()
            start = jnp.where(w_t > 0, cnt_s[jnp.maximum(w_t - 1, 0)], 0)
            end = cnt_s[w_t]

            def scan(k, carry2):
                v = sid[pl.ds(start + k * 16, 16)]
                pv = spos[pl.ds(start + k * 16, 16)]
                for jj in range(16):
                    @pl.when(start + k * 16 + jj < end)
                    def _one():
                        lane = v[jj] & (WIN - 1)
                        pos = pv[jj]
                        bc = scal[1]
                        lv = jnp.full((16,), 0, jnp.int32) + lane
                        for fb in range(EMB // 16):
                            g16 = plsc.load_gather(
                                win.at[b], [fb * 16 + iota16, lv])
                            colbuf[pl.ds(bc * EMB + fb * 16, 16)] = g16
                        posb[bc] = pos
                        scal[1] = bc + 1

                @pl.when(scal[1] >= FLUSH)
                def _fl():
                    flush_batch(scal[1])
                    scal[1] = 0
                return carry2

            lax.fori_loop(0, (end - start + 15) // 16, scan, 0)
        return carry

    lax.fori_loop(0, WPT, outer, 0)
    flush_batch(scal[1])


def _phase1_body(u_id, i_id, uT, iT, u_stage, i_stage,
                 ids_v, mid, mpos, sid, spos, win, colbuf,
                 posb, scal, cnt_s, wsem, wsems):
    c = lax.axis_index("c")
    s = lax.axis_index("s")

    @pl.when(c == 0)
    def _user():
        _phase1_sweep(u_id, uT, u_stage, ids_v, mid, mpos, sid, spos,
                      win, colbuf, posb, scal, cnt_s, wsem, wsems, s)

    @pl.when(c == 1)
    def _item():
        _phase1_sweep(i_id, iT, i_stage, ids_v, mid, mpos, sid, spos,
                      win, colbuf, posb, scal, cnt_s, wsem, wsems, s)


def _phase2_body(nw, bpw, u_id, i_id, u_stage, i_stage, ub, ib, mean, out,
                 uidx, iidx, u_v, i_v, bu, bi, mean_v, out_v, bsem):
    c = lax.axis_index("c")
    s = lax.axis_index("s")
    wid = s * 2 + c
    base = wid * bpw
    n_chunks = bpw // IDX_CHUNK

    pltpu.sync_copy(u_stage.at[pl.ds(base * EMB, bpw * EMB)], u_v)
    pltpu.sync_copy(i_stage.at[pl.ds(base * EMB, bpw * EMB)], i_v)
    for j in range(n_chunks):
        pltpu.sync_copy(u_id.at[pl.ds(base + j * IDX_CHUNK, IDX_CHUNK)], uidx.at[j])
        pltpu.sync_copy(i_id.at[pl.ds(base + j * IDX_CHUNK, IDX_CHUNK)], iidx.at[j])
    pltpu.sync_copy(mean, mean_v.at[pl.ds(0, 1)])

    bias_copies = []
    for j in range(n_chunks):
        sl = pl.ds(j * IDX_CHUNK, IDX_CHUNK)
        bias_copies.append(pltpu.async_copy(ub.at[uidx.at[j]], bu.at[sl], bsem))
        bias_copies.append(pltpu.async_copy(ib.at[iidx.at[j]], bi.at[sl], bsem))
    for cp in bias_copies:
        cp.wait()

    mean_s = mean_v[pl.ds(0, 16)][0]
    iota16 = lax.iota(jnp.int32, 16)

    def group(g, carry):
        o = g * 16
        fidx = (o + iota16) * EMB
        acc = bu[pl.ds(o, 16)] + bi[pl.ds(o, 16)] + mean_s
        for k in range(EMB):
            uu = plsc.load_gather(u_v, [fidx + k])
            ii = plsc.load_gather(i_v, [fidx + k])
            acc = acc + uu * ii
        out_v[pl.ds(o, 16)] = acc
        return carry

    lax.fori_loop(0, bpw // 16, group, 0)
    pltpu.sync_copy(out_v, out.at[pl.ds(base, bpw)])


def kernel(u_id, i_id, user_emb, user_bias, item_emb, item_bias, mean):
    info = plsc.get_sparse_core_info()
    nw = info.num_cores * info.num_subcores
    bpw = BATCH // nw
    mesh = plsc.VectorSubcoreMesh(core_axis_name="c", subcore_axis_name="s")
    params = pltpu.CompilerParams(needs_layout_passes=False,
                                  use_tc_tiling_on_sc=True)

    uid32 = u_id.astype(jnp.int32)
    iid32 = i_id.astype(jnp.int32)

    k1 = functools.partial(
        pl.kernel,
        mesh=mesh,
        out_type=(jax.ShapeDtypeStruct((BATCH * EMB,), jnp.float32),
                  jax.ShapeDtypeStruct((BATCH * EMB,), jnp.float32)),
        compiler_params=params,
        scratch_types=[
            pltpu.VMEM((BATCH,), jnp.int32),
            pltpu.VMEM((BATCH + 16,), jnp.int32),
            pltpu.VMEM((BATCH + 16,), jnp.int32),
            pltpu.VMEM((BATCH + 16,), jnp.int32),
            pltpu.VMEM((BATCH + 16,), jnp.int32),
            pltpu.VMEM((NRING, EMB, WIN), jnp.float32),
            pltpu.VMEM((CBUF * EMB,), jnp.float32),
            pltpu.SMEM((CBUF,), jnp.int32),
            pltpu.SMEM((8,), jnp.int32),
            pltpu.SMEM((WPT,), jnp.int32),
            pltpu.SemaphoreType.DMA,
            pltpu.SemaphoreType.DMA((NRING,)),
        ],
    )(_phase1_body)

    u_stage, i_stage = k1(uid32, iid32, user_emb.T, item_emb.T)

    k2 = functools.partial(
        pl.kernel,
        mesh=mesh,
        out_type=jax.ShapeDtypeStruct((BATCH,), jnp.float32),
        compiler_params=params,
        scratch_types=[
            pltpu.VMEM((bpw // IDX_CHUNK, IDX_CHUNK), jnp.int32),
            pltpu.VMEM((bpw // IDX_CHUNK, IDX_CHUNK), jnp.int32),
            pltpu.VMEM((bpw * EMB,), jnp.float32),
            pltpu.VMEM((bpw * EMB,), jnp.float32),
            pltpu.VMEM((bpw,), jnp.float32),
            pltpu.VMEM((bpw,), jnp.float32),
            pltpu.VMEM((16,), jnp.float32),
            pltpu.VMEM((bpw,), jnp.float32),
            pltpu.SemaphoreType.DMA,
        ],
    )(functools.partial(_phase2_body, nw, bpw))

    return k2(uid32, iid32, u_stage, i_stage,
              user_bias.reshape(-1), item_bias.reshape(-1), mean)


# speedup vs baseline: 1.9565x; 1.0171x over previous
import functools

import jax
import jax.numpy as jnp
from jax import lax
from jax.experimental import pallas as pl
from jax.experimental.pallas import tpu as pltpu
from jax.experimental.pallas import tpu_sc as plsc

BATCH = 16384
EMB = 64
NTAB = 1000000
IDX_CHUNK = 128
WIN = 128
NWIN = (NTAB + WIN - 1) // WIN
WPT = (NWIN + 15) // 16
FLUSH = 16
CBUF = FLUSH + 16
NRING = 4


def _phase1_sweep(ids_hbm, table, stage, ids_v, mid, mpos, sid, spos,
                  win, colbuf, posb, scal, cnt_s, wsem, wsems, t):
    first_w = t * WPT
    end_w = jnp.minimum(first_w + WPT, NWIN)
    nwin = end_w - first_w
    iota16 = lax.iota(jnp.int32, 16)
    lane0 = iota16 == 0

    pltpu.sync_copy(ids_hbm, ids_v.at[pl.ds(0, BATCH)])

    def filt(k, off):
        v = ids_v[pl.ds(k * 16, 16)]
        wv = v >> 7
        msk = (wv >= first_w) & (wv < end_w)
        cnt = plsc.all_reduce_population_count(msk)[0]
        plsc.store_compressed(mid.at[pl.ds(off, 16)], v, mask=msk)
        plsc.store_compressed(mpos.at[pl.ds(off, 16)], k * 16 + iota16, mask=msk)
        return off + cnt

    moff = lax.fori_loop(0, BATCH // 16, filt, 0)
    scal[0] = moff
    scal[1] = 0

    def czero(w, carry):
        cnt_s[w] = 0
        return carry

    lax.fori_loop(0, WPT, czero, 0)

    def count(k, carry):
        v = mid[pl.ds(k * 16, 16)]
        for jj in range(16):
            @pl.when(k * 16 + jj < moff)
            def _c():
                w = (v[jj] >> 7) - first_w
                cnt_s[w] = cnt_s[w] + 1
        return carry

    lax.fori_loop(0, (moff + 15) // 16, count, 0)

    def prefix(w, acc):
        c = cnt_s[w]
        cnt_s[w] = acc
        return acc + c

    lax.fori_loop(0, WPT, prefix, 0)

    def place(k, carry):
        v = mid[pl.ds(k * 16, 16)]
        pv = mpos[pl.ds(k * 16, 16)]
        for jj in range(16):
            @pl.when(k * 16 + jj < moff)
            def _p():
                w = (v[jj] >> 7) - first_w
                slot = cnt_s[w]
                sv = jnp.full((16,), 0, jnp.int32) + slot
                plsc.store_scatter(sid, [sv], jnp.full((16,), 0, jnp.int32) + v[jj],
                                   mask=lane0)
                plsc.store_scatter(spos, [sv], jnp.full((16,), 0, jnp.int32) + pv[jj],
                                   mask=lane0)
                cnt_s[w] = slot + 1
        return carry

    lax.fori_loop(0, (moff + 15) // 16, place, 0)

    def flush_batch(n_cond):
        for q in range(CBUF):
            @pl.when(q < n_cond)
            def _():
                p = posb[q]
                pltpu.async_copy(colbuf.at[pl.ds(q * EMB, EMB)],
                                 stage.at[pl.ds(p * EMB, EMB)], wsem)
        for q in range(CBUF):
            @pl.when(q < n_cond)
            def _():
                pltpu.make_async_copy(colbuf.at[pl.ds(0, EMB)],
                                      stage.at[pl.ds(0, EMB)], wsem).wait()

    def nonempty(w_t):
        start = jnp.where(w_t > 0, cnt_s[jnp.maximum(w_t - 1, 0)], 0)
        return cnt_s[jnp.minimum(w_t, WPT - 1)] > start

    def fire(w_t):
        @pl.when((w_t < nwin) & nonempty(w_t))
        def _():
            slot = w_t % NRING
            col0 = pl.multiple_of((first_w + w_t) * WIN, WIN)
            pltpu.async_copy(table.at[:, pl.ds(col0, WIN)], win.at[slot],
                             wsems.at[slot])

    for p in range(NRING - 1):
        fire(p)

    def outer(w, carry):
        fire(w + NRING - 1)
        w_t = w
        b = w_t % NRING

        @pl.when((w_t < nwin) & nonempty(w_t))
        def _process():
            pltpu.make_async_copy(table.at[:, pl.ds(0, WIN)],
                                  win.at[b], wsems.at[b]).wait()
            start = jnp.where(w_t > 0, cnt_s[jnp.maximum(w_t - 1, 0)], 0)
            end = cnt_s[w_t]

            def scan(k, carry2):
                v = sid[pl.ds(start + k * 16, 16)]
                pv = spos[pl.ds(start + k * 16, 16)]
                for jj in range(16):
                    @pl.when(start + k * 16 + jj < end)
                    def _one():
                        lane = v[jj] & (WIN - 1)
                        pos = pv[jj]
                        bc = scal[1]
                        lv = jnp.full((16,), 0, jnp.int32) + lane
                        for fb in range(EMB // 16):
                            g16 = plsc.load_gather(
                                win.at[b], [fb * 16 + iota16, lv])
                            colbuf[pl.ds(bc * EMB + fb * 16, 16)] = g16
                        posb[bc] = pos
                        scal[1] = bc + 1

                @pl.when(scal[1] >= FLUSH)
                def _fl():
                    flush_batch(scal[1])
                    scal[1] = 0
                return carry2

            lax.fori_loop(0, (end - start + 15) // 16, scan, 0)
        return carry

    lax.fori_loop(0, WPT, outer, 0)
    flush_batch(scal[1])


def _phase1_body(u_id, i_id, uT, iT, u_stage, i_stage,
                 ids_v, mid, mpos, sid, spos, win, colbuf,
                 posb, scal, cnt_s, wsem, wsems):
    c = lax.axis_index("c")
    s = lax.axis_index("s")

    @pl.when(c == 0)
    def _user():
        _phase1_sweep(u_id, uT, u_stage, ids_v, mid, mpos, sid, spos,
                      win, colbuf, posb, scal, cnt_s, wsem, wsems, s)

    @pl.when(c == 1)
    def _item():
        _phase1_sweep(i_id, iT, i_stage, ids_v, mid, mpos, sid, spos,
                      win, colbuf, posb, scal, cnt_s, wsem, wsems, s)


def _phase2_body(nw, bpw, u_id, i_id, u_stage, i_stage, ub, ib, mean, out,
                 uidx, iidx, u_v, i_v, bu, bi, mean_v, out_v, bsem):
    c = lax.axis_index("c")
    s = lax.axis_index("s")
    wid = s * 2 + c
    base = wid * bpw
    n_chunks = bpw // IDX_CHUNK

    pltpu.sync_copy(u_stage.at[pl.ds(base * EMB, bpw * EMB)], u_v)
    pltpu.sync_copy(i_stage.at[pl.ds(base * EMB, bpw * EMB)], i_v)
    for j in range(n_chunks):
        pltpu.sync_copy(u_id.at[pl.ds(base + j * IDX_CHUNK, IDX_CHUNK)], uidx.at[j])
        pltpu.sync_copy(i_id.at[pl.ds(base + j * IDX_CHUNK, IDX_CHUNK)], iidx.at[j])
    pltpu.sync_copy(mean, mean_v.at[pl.ds(0, 1)])

    bias_copies = []
    for j in range(n_chunks):
        sl = pl.ds(j * IDX_CHUNK, IDX_CHUNK)
        bias_copies.append(pltpu.async_copy(ub.at[uidx.at[j]], bu.at[sl], bsem))
        bias_copies.append(pltpu.async_copy(ib.at[iidx.at[j]], bi.at[sl], bsem))
    for cp in bias_copies:
        cp.wait()

    mean_s = mean_v[pl.ds(0, 16)][0]
    iota16 = lax.iota(jnp.int32, 16)

    def group(g, carry):
        o = g * 16
        fidx = (o + iota16) * EMB
        acc = bu[pl.ds(o, 16)] + bi[pl.ds(o, 16)] + mean_s
        for k in range(EMB):
            uu = plsc.load_gather(u_v, [fidx + k])
            ii = plsc.load_gather(i_v, [fidx + k])
            acc = acc + uu * ii
        out_v[pl.ds(o, 16)] = acc
        return carry

    lax.fori_loop(0, bpw // 16, group, 0)
    pltpu.sync_copy(out_v, out.at[pl.ds(base, bpw)])


def kernel(u_id, i_id, user_emb, user_bias, item_emb, item_bias, mean):
    info = plsc.get_sparse_core_info()
    nw = info.num_cores * info.num_subcores
    bpw = BATCH // nw
    mesh = plsc.VectorSubcoreMesh(core_axis_name="c", subcore_axis_name="s")
    params = pltpu.CompilerParams(needs_layout_passes=False,
                                  use_tc_tiling_on_sc=True)

    uid32 = u_id.astype(jnp.int32)
    iid32 = i_id.astype(jnp.int32)

    k1 = functools.partial(
        pl.kernel,
        mesh=mesh,
        out_type=(jax.ShapeDtypeStruct((BATCH * EMB,), jnp.float32),
                  jax.ShapeDtypeStruct((BATCH * EMB,), jnp.float32)),
        compiler_params=params,
        scratch_types=[
            pltpu.VMEM((BATCH,), jnp.int32),
            pltpu.VMEM((BATCH + 16,), jnp.int32),
            pltpu.VMEM((BATCH + 16,), jnp.int32),
            pltpu.VMEM((BATCH + 16,), jnp.int32),
            pltpu.VMEM((BATCH + 16,), jnp.int32),
            pltpu.VMEM((NRING, EMB, WIN), jnp.float32),
            pltpu.VMEM((CBUF * EMB,), jnp.float32),
            pltpu.SMEM((CBUF,), jnp.int32),
            pltpu.SMEM((8,), jnp.int32),
            pltpu.SMEM((WPT,), jnp.int32),
            pltpu.SemaphoreType.DMA,
            pltpu.SemaphoreType.DMA((NRING,)),
        ],
    )(_phase1_body)

    u_stage, i_stage = k1(uid32, iid32, user_emb.T, item_emb.T)

    k2 = functools.partial(
        pl.kernel,
        mesh=mesh,
        out_type=jax.ShapeDtypeStruct((BATCH,), jnp.float32),
        compiler_params=params,
        scratch_types=[
            pltpu.VMEM((bpw // IDX_CHUNK, IDX_CHUNK), jnp.int32),
            pltpu.VMEM((bpw // IDX_CHUNK, IDX_CHUNK), jnp.int32),
            pltpu.VMEM((bpw * EMB,), jnp.float32),
            pltpu.VMEM((bpw * EMB,), jnp.float32),
            pltpu.VMEM((bpw,), jnp.float32),
            pltpu.VMEM((bpw,), jnp.float32),
            pltpu.VMEM((16,), jnp.float32),
            pltpu.VMEM((bpw,), jnp.float32),
            pltpu.SemaphoreType.DMA,
        ],
    )(functools.partial(_phase2_body, nw, bpw))

    return k2(uid32, iid32, u_stage, i_stage,
              user_bias.reshape(-1), item_bias.reshape(-1), mean)
